# Initial kernel scaffold; baseline (speedup 1.0000x reference)
import jax
import jax.numpy as jnp
from jax.experimental import pallas as pl


def kernel(source_control_points, inverse_kernel, target_coordinate_repr, target_coordinate_origin, padding_matrix):
    raise NotImplementedError("write your pallas kernel here")


# speedup vs baseline: 50.2053x; 50.2053x over previous
import functools

import jax
import jax.numpy as jnp
from jax import lax
from jax.experimental import pallas as pl
from jax.experimental.pallas import tpu as pltpu
from jax.experimental.pallas import tpu_sc as plsc

W = 512
H = 512
HW = H * W
B = 16
K = 103

ROWS = 2048
GRID = HW // ROWS

NC = 2
NS = 16
NW = NC * NS
NQ = 4
QW = HW // NQ
TASKS_PER_WORKER = (B * NQ) // NW
CH = 16384
NCH = HW // CH


def _tc_body(inv_ref, yf_ref, tcr_ref, sct_ref, idx_ref):
    m = jnp.dot(inv_ref[...], yf_ref[...], preferred_element_type=jnp.float32)
    st = lax.dot_general(m, tcr_ref[...], (((0,), (1,)), ((), ())),
                         preferred_element_type=jnp.float32)
    sct_ref[...] = st
    x = st[:B, :]
    y = st[B:, :]
    fx = (x + 1.0) * 256.0
    fy = (y + 1.0) * 256.0
    fsum = fx + 512.0 * fy
    idx_ref[...] = fsum.astype(jnp.int32)


def _tc_compute(inv, yf, tcr):
    return pl.pallas_call(
        _tc_body,
        grid=(GRID,),
        in_specs=[
            pl.BlockSpec((K, K), lambda i: (0, 0)),
            pl.BlockSpec((K, 2 * B), lambda i: (0, 0)),
            pl.BlockSpec((ROWS, K), lambda i: (i, 0)),
        ],
        out_specs=[
            pl.BlockSpec((2 * B, ROWS), lambda i: (0, i)),
            pl.BlockSpec((B, ROWS), lambda i: (0, i)),
        ],
        out_shape=[
            jax.ShapeDtypeStruct((2 * B, HW), jnp.float32),
            jax.ShapeDtypeStruct((B, HW), jnp.int32),
        ],
    )(inv, yf, tcr)


@functools.cache
def _sc_scatter_kernel():
    mesh = plsc.VectorSubcoreMesh(core_axis_name="c", subcore_axis_name="s",
                                  num_cores=NC, num_subcores=NS)
    return pl.kernel(
        _sc_scatter_body,
        out_type=jax.ShapeDtypeStruct((B, HW), jnp.float32),
        mesh=mesh,
        scratch_types=[
            pltpu.VMEM((QW,), jnp.float32),
            pltpu.VMEM((CH,), jnp.int32),
        ],
        compiler_params=pltpu.CompilerParams(needs_layout_passes=False),
    )


def _sc_scatter_body(idx_hbm, out_hbm, mmap, ibuf):
    wid = lax.axis_index("s") * NC + lax.axis_index("c")
    zero16 = jnp.zeros((16,), jnp.float32)

    for t in range(TASKS_PER_WORKER):
        tid = wid * TASKS_PER_WORKER + t
        b = tid // NQ
        q = tid % NQ
        qbase = q * QW

        def zbody(i, c):
            mmap[pl.ds(i * 16, 16)] = zero16
            return c
        lax.fori_loop(0, QW // 16, zbody, 0, unroll=8)

        def chunk_body(ci, c):
            pltpu.sync_copy(idx_hbm.at[b, pl.ds(ci * CH, CH)], ibuf)
            chunk_row0 = (ci * CH) // W

            def vbody(v, c2):
                iv = ibuf[pl.ds(v * 16, 16)]
                local = iv - qbase
                msk = plsc.bitcast(local, jnp.uint32) < QW
                row = chunk_row0 + v // (W // 16)
                val = jnp.full((16,), 1.0, jnp.float32) * row.astype(jnp.float32)
                plsc.store_scatter(mmap, [local], val, mask=msk)
                return c2
            lax.fori_loop(0, CH // 16, vbody, 0, unroll=8)
            return c
        lax.fori_loop(0, NCH, chunk_body, 0)

        pltpu.sync_copy(mmap, out_hbm.at[b, pl.ds(qbase, QW)])


def kernel(source_control_points, inverse_kernel, target_coordinate_repr,
           target_coordinate_origin, padding_matrix):
    bn = source_control_points.shape[0]
    y = jnp.concatenate(
        [source_control_points, jnp.broadcast_to(padding_matrix, (bn, 3, 2))],
        axis=1)
    yf = y.transpose(1, 2, 0).reshape(K, 2 * bn)

    sct, idx = _tc_compute(inverse_kernel, yf, target_coordinate_repr)
    source_coordinate = sct.reshape(2, bn, HW).transpose(1, 2, 0)

    m = _sc_scatter_kernel()(idx)
    map_x = m.reshape(bn, H, W)
    return (source_coordinate, map_x, map_x)


# speedup vs baseline: 101.7079x; 2.0258x over previous
import functools

import jax
import jax.numpy as jnp
from jax import lax
from jax.experimental import pallas as pl
from jax.experimental.pallas import tpu as pltpu
from jax.experimental.pallas import tpu_sc as plsc

W = 512
H = 512
HW = H * W
B = 16
K = 103

NSPLIT = 2
ROWS = 16384
STEP = NSPLIT * ROWS
GRID = HW // STEP

NC = 2
NS = 16
NW = NC * NS
NQ = 4
QW = HW // NQ
TASKS_PER_WORKER = (B * NQ) // NW
CH = 16384
NCH = HW // CH


def _tc_body(inv_ref, yf_ref, *refs):
    tcr_refs = refs[:NSPLIT]
    sct_ref, idx_ref = refs[NSPLIT:]
    m = jnp.dot(inv_ref[...], yf_ref[...], preferred_element_type=jnp.float32)
    for j in range(NSPLIT):
        st = lax.dot_general(m, tcr_refs[j][...], (((0,), (1,)), ((), ())),
                             preferred_element_type=jnp.float32)
        x = st[:B, :]
        y = st[B:, :]
        sct_ref[:, j * ROWS:(j + 1) * ROWS] = st
        fx = (x + 1.0) * 256.0
        fy = (y + 1.0) * 256.0
        fsum = fx + 512.0 * fy
        idx_ref[:, j * ROWS:(j + 1) * ROWS] = fsum.astype(jnp.int32)


def _tc_compute(inv, yf, tcr):
    def tcr_spec(j):
        return pl.BlockSpec((ROWS, K), lambda i, j=j: (NSPLIT * i + j, 0))
    return pl.pallas_call(
        _tc_body,
        grid=(GRID,),
        in_specs=[
            pl.BlockSpec((K, K), lambda i: (0, 0)),
            pl.BlockSpec((K, 2 * B), lambda i: (0, 0)),
        ] + [tcr_spec(j) for j in range(NSPLIT)],
        out_specs=[
            pl.BlockSpec((2 * B, STEP), lambda i: (0, i)),
            pl.BlockSpec((B, STEP), lambda i: (0, i)),
        ],
        out_shape=[
            jax.ShapeDtypeStruct((2 * B, HW), jnp.float32),
            jax.ShapeDtypeStruct((B, HW), jnp.int32),
        ],
    )(inv, yf, *([tcr] * NSPLIT))


@functools.cache
def _sc_scatter_kernel():
    mesh = plsc.VectorSubcoreMesh(core_axis_name="c", subcore_axis_name="s",
                                  num_cores=NC, num_subcores=NS)
    return pl.kernel(
        _sc_scatter_body,
        out_type=jax.ShapeDtypeStruct((B, HW), jnp.float32),
        mesh=mesh,
        scratch_types=[
            pltpu.VMEM((QW,), jnp.float32),
            pltpu.VMEM((CH,), jnp.int32),
            pltpu.VMEM((CH,), jnp.int32),
            pltpu.SemaphoreType.DMA,
            pltpu.SemaphoreType.DMA,
        ],
        compiler_params=pltpu.CompilerParams(needs_layout_passes=False),
    )


def _sc_scatter_body(idx_hbm, out_hbm, mmap, ibuf0, ibuf1, sem0, sem1):
    wid = lax.axis_index("s") * NC + lax.axis_index("c")
    zero16 = jnp.zeros((16,), jnp.float32)
    bufs = (ibuf0, ibuf1)
    sems = (sem0, sem1)
    vec_per_row = W // 16
    rows_per_chunk = CH // W

    for t in range(TASKS_PER_WORKER):
        tid = wid * TASKS_PER_WORKER + t
        b = tid // NQ
        q = tid % NQ
        qbase = q * QW

        @plsc.parallel_loop(0, QW // 16, unroll=8)
        def _zero(i):
            mmap[pl.ds(i * 16, 16)] = zero16

        pltpu.async_copy(idx_hbm.at[b, pl.ds(0, CH)], ibuf0, sem0)

        def process_chunk(ci, buf):
            chunk_row0 = (ci * CH) // W

            def rbody(r, c):
                val = jnp.full((16,), 1.0, jnp.float32) \
                    * (chunk_row0 + r).astype(jnp.float32)

                @plsc.parallel_loop(0, vec_per_row, unroll=8)
                def _scatter_row(v):
                    iv = buf[pl.ds((r * vec_per_row + v) * 16, 16)]
                    local = iv - qbase
                    msk = plsc.bitcast(local, jnp.uint32) < QW
                    plsc.store_scatter(mmap, [local], val, mask=msk)
                return c
            lax.fori_loop(0, rows_per_chunk, rbody, 0)

        def chunk_pair(cp, c):
            for k in range(2):
                ci = cp * 2 + k
                pltpu.make_async_copy(
                    idx_hbm.at[b, pl.ds(ci * CH, CH)], bufs[k], sems[k]).wait()

                @pl.when(ci + 1 < NCH)
                def _():
                    pltpu.async_copy(
                        idx_hbm.at[b, pl.ds((ci + 1) * CH, CH)],
                        bufs[1 - k], sems[1 - k])
                process_chunk(ci, bufs[k])
            return c
        lax.fori_loop(0, NCH // 2, chunk_pair, 0)

        pltpu.sync_copy(mmap, out_hbm.at[b, pl.ds(qbase, QW)])


def kernel(source_control_points, inverse_kernel, target_coordinate_repr,
           target_coordinate_origin, padding_matrix):
    bn = source_control_points.shape[0]
    y = jnp.concatenate(
        [source_control_points, jnp.broadcast_to(padding_matrix, (bn, 3, 2))],
        axis=1)
    yf = y.transpose(1, 2, 0).reshape(K, 2 * bn)

    sct, idx = _tc_compute(inverse_kernel, yf, target_coordinate_repr)
    source_coordinate = sct.reshape(2, bn, HW).transpose(1, 2, 0)

    m = _sc_scatter_kernel()(idx)
    map_x = m.reshape(bn, H, W)
    return (source_coordinate, map_x, map_x)


# speedup vs baseline: 101.8344x; 1.0012x over previous
import functools

import jax
import jax.numpy as jnp
from jax import lax
from jax.experimental import pallas as pl
from jax.experimental.pallas import tpu as pltpu
from jax.experimental.pallas import tpu_sc as plsc

W = 512
H = 512
HW = H * W
B = 16
K = 103

NSPLIT = 1
ROWS = 16384
STEP = NSPLIT * ROWS
GRID = HW // STEP

NC = 2
NS = 16
NW = NC * NS
NQ = 4
QW = HW // NQ
TASKS_PER_WORKER = (B * NQ) // NW
CH = 16384
NCH = HW // CH


def _tc_body(inv_ref, yf_ref, *refs):
    tcr_refs = refs[:NSPLIT]
    sct_ref, idx_ref = refs[NSPLIT:]
    m = jnp.dot(inv_ref[...], yf_ref[...], preferred_element_type=jnp.float32)
    for j in range(NSPLIT):
        st = lax.dot_general(m, tcr_refs[j][...], (((0,), (1,)), ((), ())),
                             preferred_element_type=jnp.float32)
        x = st[:B, :]
        y = st[B:, :]
        sct_ref[:, j * ROWS:(j + 1) * ROWS] = st
        fx = (x + 1.0) * 256.0
        fy = (y + 1.0) * 256.0
        fsum = fx + 512.0 * fy
        idx_ref[:, j * ROWS:(j + 1) * ROWS] = fsum.astype(jnp.int32)


def _tc_compute(inv, yf, tcr):
    def tcr_spec(j):
        return pl.BlockSpec((ROWS, K), lambda i, j=j: (NSPLIT * i + j, 0))
    return pl.pallas_call(
        _tc_body,
        grid=(GRID,),
        in_specs=[
            pl.BlockSpec((K, K), lambda i: (0, 0)),
            pl.BlockSpec((K, 2 * B), lambda i: (0, 0)),
        ] + [tcr_spec(j) for j in range(NSPLIT)],
        out_specs=[
            pl.BlockSpec((2 * B, STEP), lambda i: (0, i)),
            pl.BlockSpec((B, STEP), lambda i: (0, i)),
        ],
        out_shape=[
            jax.ShapeDtypeStruct((2 * B, HW), jnp.float32),
            jax.ShapeDtypeStruct((B, HW), jnp.int32),
        ],
    )(inv, yf, *([tcr] * NSPLIT))


@functools.cache
def _sc_scatter_kernel():
    mesh = plsc.VectorSubcoreMesh(core_axis_name="c", subcore_axis_name="s",
                                  num_cores=NC, num_subcores=NS)
    return pl.kernel(
        _sc_scatter_body,
        out_type=jax.ShapeDtypeStruct((B, HW), jnp.float32),
        mesh=mesh,
        scratch_types=[
            pltpu.VMEM((QW,), jnp.float32),
            pltpu.VMEM((CH,), jnp.int32),
            pltpu.VMEM((CH,), jnp.int32),
            pltpu.SemaphoreType.DMA,
            pltpu.SemaphoreType.DMA,
        ],
        compiler_params=pltpu.CompilerParams(needs_layout_passes=False),
    )


def _sc_scatter_body(idx_hbm, out_hbm, mmap, ibuf0, ibuf1, sem0, sem1):
    wid = lax.axis_index("s") * NC + lax.axis_index("c")
    zero16 = jnp.zeros((16,), jnp.float32)
    bufs = (ibuf0, ibuf1)
    sems = (sem0, sem1)
    vec_per_row = W // 16
    rows_per_chunk = CH // W

    for t in range(TASKS_PER_WORKER):
        tid = wid * TASKS_PER_WORKER + t
        b = tid // NQ
        q = tid % NQ
        qbase = q * QW

        @plsc.parallel_loop(0, QW // 16, unroll=8)
        def _zero(i):
            mmap[pl.ds(i * 16, 16)] = zero16

        pltpu.async_copy(idx_hbm.at[b, pl.ds(0, CH)], ibuf0, sem0)

        def process_chunk(ci, buf):
            chunk_row0 = (ci * CH) // W

            def rbody(r, c):
                val = jnp.full((16,), 1.0, jnp.float32) \
                    * (chunk_row0 + r).astype(jnp.float32)

                @plsc.parallel_loop(0, vec_per_row, unroll=8)
                def _scatter_row(v):
                    iv = buf[pl.ds((r * vec_per_row + v) * 16, 16)]
                    local = iv - qbase
                    msk = plsc.bitcast(local, jnp.uint32) < QW
                    plsc.store_scatter(mmap, [local], val, mask=msk)
                return c
            lax.fori_loop(0, rows_per_chunk, rbody, 0)

        def chunk_pair(cp, c):
            for k in range(2):
                ci = cp * 2 + k
                pltpu.make_async_copy(
                    idx_hbm.at[b, pl.ds(ci * CH, CH)], bufs[k], sems[k]).wait()

                @pl.when(ci + 1 < NCH)
                def _():
                    pltpu.async_copy(
                        idx_hbm.at[b, pl.ds((ci + 1) * CH, CH)],
                        bufs[1 - k], sems[1 - k])
                process_chunk(ci, bufs[k])
            return c
        lax.fori_loop(0, NCH // 2, chunk_pair, 0)

        pltpu.sync_copy(mmap, out_hbm.at[b, pl.ds(qbase, QW)])


def kernel(source_control_points, inverse_kernel, target_coordinate_repr,
           target_coordinate_origin, padding_matrix):
    bn = source_control_points.shape[0]
    y = jnp.concatenate(
        [source_control_points, jnp.broadcast_to(padding_matrix, (bn, 3, 2))],
        axis=1)
    yf = y.transpose(1, 2, 0).reshape(K, 2 * bn)

    sct, idx = _tc_compute(inverse_kernel, yf, target_coordinate_repr)
    source_coordinate = sct.reshape(2, bn, HW).transpose(1, 2, 0)

    m = _sc_scatter_kernel()(idx)
    map_x = m.reshape(bn, H, W)
    return (source_coordinate, map_x, map_x)
